# baseline (speedup 1.0000x reference)
import functools

import jax
import jax.numpy as jnp
from jax import lax
from jax.experimental import pallas as pl
from jax.experimental.pallas import tpu as pltpu
from jax.experimental.pallas import tpu_sc as plsc

_K = 1024
_D = 256
_C = 192
_B = 32
_BETA = 0.25

_CPB = 16


def _vq_body(z_ref, e_ref, idx_ref, flat_ref, loss_ref):
    step = pl.program_id(0)

    @pl.when(step == 0)
    def _():
        loss_ref[0, 0] = 0.0

    acc = 0.0
    for j in range(_CPB):
        zf = z_ref[:, j, :]
        e = e_ref[j]
        z2 = jnp.sum(zf * zf, axis=-1, keepdims=True)
        e2 = jnp.sum(e * e, axis=-1)
        m = jax.lax.dot_general(zf, e, (((1,), (1,)), ((), ())),
                                preferred_element_type=jnp.float32)
        dist = z2 - 2.0 * m + e2[None, :]
        idx = jnp.argmin(dist, axis=-1).astype(jnp.int32)
        idx_ref[j, 0, :] = idx
        flat_ref[j, 0, :] = idx + (step * _CPB + j) * _K
        acc = acc + jnp.sum(jnp.min(dist, axis=-1))

    loss_ref[0, 0] += acc

    @pl.when(step == (_C // _CPB) - 1)
    def _():
        loss_ref[0, 0] = loss_ref[0, 0] * (_BETA / (_B * _C * _D))


@functools.partial(jax.jit, static_argnames=("interpret",))
def _vq_tc(z_flat, embedding, interpret=False):
    idx3, flat3, loss = pl.pallas_call(
        _vq_body,
        grid=(_C // _CPB,),
        in_specs=[
            pl.BlockSpec((_B, _CPB, _D), lambda c: (0, c, 0)),
            pl.BlockSpec((_CPB, _K, _D), lambda c: (c, 0, 0)),
        ],
        out_specs=[
            pl.BlockSpec((_CPB, 1, _B), lambda c: (c, 0, 0)),
            pl.BlockSpec((_CPB, 1, _B), lambda c: (c, 0, 0)),
            pl.BlockSpec(memory_space=pltpu.SMEM, block_shape=(1, 1),
                         index_map=lambda c: (0, 0)),
        ],
        out_shape=[
            jax.ShapeDtypeStruct((_C, 1, _B), jnp.int32),
            jax.ShapeDtypeStruct((_C, 1, _B), jnp.int32),
            jax.ShapeDtypeStruct((1, 1), jnp.float32),
        ],
        interpret=interpret,
    )(z_flat, embedding)
    return idx3, flat3, loss


def _make_sc_gather():
    info = plsc.get_sparse_core_info()
    nw = info.num_cores * info.num_subcores
    rows = _B * _C
    rpw = rows // nw
    nch = 2
    cpw = rpw // nch
    mesh = plsc.VectorSubcoreMesh(core_axis_name="c", subcore_axis_name="s")

    @functools.partial(
        pl.kernel, mesh=mesh,
        out_type=jax.ShapeDtypeStruct((rows, _D), jnp.float32),
        scratch_types=[
            pltpu.VMEM((nch, cpw), jnp.int32),
            pltpu.VMEM((cpw, _D), jnp.float32),
            pltpu.VMEM((cpw, _D), jnp.float32),
            pltpu.SemaphoreType.DMA,
            pltpu.SemaphoreType.DMA,
        ],
    )
    def sc_gather(table_hbm, idx_hbm, out_hbm, idx_v, rows_a, rows_b, sem_a, sem_b):
        wid = lax.axis_index("s") * info.num_cores + lax.axis_index("c")
        pltpu.sync_copy(idx_hbm.at[wid], idx_v)
        base = wid * rpw
        cp_a = pltpu.async_copy(table_hbm.at[idx_v.at[0]], rows_a, sem_a)
        cp_b = pltpu.async_copy(table_hbm.at[idx_v.at[1]], rows_b, sem_b)
        cp_a.wait()
        pltpu.sync_copy(rows_a, out_hbm.at[pl.ds(base, cpw)])
        cp_b.wait()
        pltpu.sync_copy(rows_b, out_hbm.at[pl.ds(base + cpw, cpw)])

    return sc_gather, nw, nch, cpw


def kernel(z, embedding):
    b, c, h, w = z.shape
    d = h * w
    z_flat = z.reshape(b, c, d)
    idx3, flat3, loss = _vq_tc(z_flat, embedding)
    indices_out = idx3.reshape(c, b).transpose(1, 0)
    commit_loss = loss[0, 0]

    sc_gather, nw, nch, cpw = _make_sc_gather()
    table = embedding.reshape(c * _K, d)
    flat_idx = flat3.reshape(c, b).transpose(1, 0).reshape(nw, nch, cpw)
    q_rows = sc_gather(table, flat_idx)
    q_st = q_rows.reshape(b, c, h, w)
    return (q_st, commit_loss, indices_out)


# speedup vs baseline: 1.0765x; 1.0765x over previous
import functools

import jax
import jax.numpy as jnp
from jax import lax
from jax.experimental import pallas as pl
from jax.experimental.pallas import tpu as pltpu
from jax.experimental.pallas import tpu_sc as plsc

_K = 1024
_D = 256
_C = 192
_B = 32
_BETA = 0.25

_CPB = 8


def _vq_body(z_ref, e_ref, idx_ref, flat_ref, loss_ref):
    step = pl.program_id(0)

    acc = 0.0
    for j in range(_CPB):
        zf = z_ref[:, j, :]
        e = e_ref[j]
        z2 = jnp.sum(zf * zf, axis=-1, keepdims=True)
        e2 = jnp.sum(e * e, axis=-1)
        m = jax.lax.dot_general(zf, e, (((1,), (1,)), ((), ())),
                                preferred_element_type=jnp.float32)
        e2l = e2.reshape(_K // 128, 128)
        dist = jnp.concatenate(
            [z2 - 2.0 * m[:, ci * 128:(ci + 1) * 128] + e2l[ci][None, :]
             for ci in range(_K // 128)], axis=1)
        idx = jnp.argmin(dist, axis=-1).astype(jnp.int32)
        idx_ref[j, 0, :] = idx
        flat_ref[j, 0, :] = idx + (step * _CPB + j) * _K
        acc = acc + jnp.sum(jnp.min(dist, axis=-1))

    loss_ref[0, 0, 0] = acc


@functools.partial(jax.jit, static_argnames=("interpret",))
def _vq_tc(z_flat, embedding, interpret=False):
    idx3, flat3, loss = pl.pallas_call(
        _vq_body,
        grid=(_C // _CPB,),
        in_specs=[
            pl.BlockSpec((_B, _CPB, _D), lambda c: (0, c, 0)),
            pl.BlockSpec((_CPB, _K, _D), lambda c: (c, 0, 0)),
        ],
        out_specs=[
            pl.BlockSpec((_CPB, 1, _B), lambda c: (c, 0, 0)),
            pl.BlockSpec((_CPB, 1, _B), lambda c: (c, 0, 0)),
            pl.BlockSpec(memory_space=pltpu.SMEM, block_shape=(1, 1, 1),
                         index_map=lambda c: (c, 0, 0)),
        ],
        out_shape=[
            jax.ShapeDtypeStruct((_C, 1, _B), jnp.int32),
            jax.ShapeDtypeStruct((_C, 1, _B), jnp.int32),
            jax.ShapeDtypeStruct((_C // _CPB, 1, 1), jnp.float32),
        ],
        compiler_params=pltpu.CompilerParams(
            dimension_semantics=("parallel",)),
        interpret=interpret,
    )(z_flat, embedding)
    return idx3, flat3, loss


def _make_sc_gather():
    info = plsc.get_sparse_core_info()
    nw = info.num_cores * info.num_subcores
    rows = _B * _C
    rpw = rows // nw
    nch = 2
    cpw = rpw // nch
    mesh = plsc.VectorSubcoreMesh(core_axis_name="c", subcore_axis_name="s")

    @functools.partial(
        pl.kernel, mesh=mesh,
        out_type=jax.ShapeDtypeStruct((rows, _D), jnp.float32),
        scratch_types=[
            pltpu.VMEM((nch, cpw), jnp.int32),
            pltpu.VMEM((cpw, _D), jnp.float32),
            pltpu.VMEM((cpw, _D), jnp.float32),
            pltpu.SemaphoreType.DMA,
            pltpu.SemaphoreType.DMA,
        ],
    )
    def sc_gather(table_hbm, idx_hbm, out_hbm, idx_v, rows_a, rows_b, sem_a, sem_b):
        wid = lax.axis_index("s") * info.num_cores + lax.axis_index("c")
        pltpu.sync_copy(idx_hbm.at[wid], idx_v)
        base = wid * rpw
        cp_a = pltpu.async_copy(table_hbm.at[idx_v.at[0]], rows_a, sem_a)
        cp_b = pltpu.async_copy(table_hbm.at[idx_v.at[1]], rows_b, sem_b)
        cp_a.wait()
        pltpu.sync_copy(rows_a, out_hbm.at[pl.ds(base, cpw)])
        cp_b.wait()
        pltpu.sync_copy(rows_b, out_hbm.at[pl.ds(base + cpw, cpw)])

    return sc_gather, nw, nch, cpw


def kernel(z, embedding):
    b, c, h, w = z.shape
    d = h * w
    z_flat = z.reshape(b, c, d)
    idx3, flat3, loss = _vq_tc(z_flat, embedding)
    indices_out = idx3.reshape(c, b).transpose(1, 0)
    commit_loss = jnp.sum(loss) * (_BETA / (_B * _C * _D))

    sc_gather, nw, nch, cpw = _make_sc_gather()
    table = embedding.reshape(c * _K, d)
    flat_idx = flat3.reshape(c, b).transpose(1, 0).reshape(nw, nch, cpw)
    q_rows = sc_gather(table, flat_idx)
    q_st = q_rows.reshape(b, c, h, w)
    return (q_st, commit_loss, indices_out)


# speedup vs baseline: 1.1316x; 1.0511x over previous
import functools

import jax
import jax.numpy as jnp
from jax import lax
from jax.experimental import pallas as pl
from jax.experimental.pallas import tpu as pltpu
from jax.experimental.pallas import tpu_sc as plsc

_K = 1024
_D = 256
_C = 192
_B = 32
_BETA = 0.25

_CPB = 8


def _vq_body(z_ref, e_ref, idx_ref, flat_ref, loss_ref):
    step = pl.program_id(0)

    acc = jnp.zeros((_B,), jnp.float32)
    for j in range(_CPB):
        zf = z_ref[:, j, :]
        e = e_ref[j]
        z2 = jnp.sum(zf * zf, axis=-1, keepdims=True)
        e2 = jnp.sum(e * e, axis=-1)
        m = jax.lax.dot_general(zf, e, (((1,), (1,)), ((), ())),
                                preferred_element_type=jnp.float32)
        e2l = e2.reshape(_K // 128, 128)
        dist = jnp.concatenate(
            [z2 - 2.0 * m[:, ci * 128:(ci + 1) * 128] + e2l[ci][None, :]
             for ci in range(_K // 128)], axis=1)
        idx = jnp.argmin(dist, axis=-1).astype(jnp.int32)
        idx_ref[j, 0, :] = idx
        flat_ref[j, 0, :] = idx + (step * _CPB + j) * _K
        acc = acc + jnp.min(dist, axis=-1)

    loss_ref[0, 0, :] = acc


@functools.partial(jax.jit, static_argnames=("interpret",))
def _vq_tc(z_flat, embedding, interpret=False):
    idx3, flat3, loss = pl.pallas_call(
        _vq_body,
        grid=(_C // _CPB,),
        in_specs=[
            pl.BlockSpec((_B, _CPB, _D), lambda c: (0, c, 0)),
            pl.BlockSpec((_CPB, _K, _D), lambda c: (c, 0, 0)),
        ],
        out_specs=[
            pl.BlockSpec((_CPB, 1, _B), lambda c: (c, 0, 0)),
            pl.BlockSpec((_CPB, 1, _B), lambda c: (c, 0, 0)),
            pl.BlockSpec((1, 1, _B), lambda c: (c, 0, 0)),
        ],
        out_shape=[
            jax.ShapeDtypeStruct((_C, 1, _B), jnp.int32),
            jax.ShapeDtypeStruct((_C, 1, _B), jnp.int32),
            jax.ShapeDtypeStruct((_C // _CPB, 1, _B), jnp.float32),
        ],
        compiler_params=pltpu.CompilerParams(
            dimension_semantics=("parallel",)),
        interpret=interpret,
    )(z_flat, embedding)
    return idx3, flat3, loss


def _make_sc_gather():
    info = plsc.get_sparse_core_info()
    nw = info.num_cores * info.num_subcores
    rows = _B * _C
    rpw = rows // nw
    nch = 2
    cpw = rpw // nch
    mesh = plsc.VectorSubcoreMesh(core_axis_name="c", subcore_axis_name="s")

    @functools.partial(
        pl.kernel, mesh=mesh,
        out_type=jax.ShapeDtypeStruct((rows, _D), jnp.float32),
        scratch_types=[
            pltpu.VMEM((nch, cpw), jnp.int32),
            pltpu.VMEM((cpw, _D), jnp.float32),
            pltpu.VMEM((cpw, _D), jnp.float32),
            pltpu.SemaphoreType.DMA,
            pltpu.SemaphoreType.DMA,
        ],
    )
    def sc_gather(table_hbm, idx_hbm, out_hbm, idx_v, rows_a, rows_b, sem_a, sem_b):
        wid = lax.axis_index("s") * info.num_cores + lax.axis_index("c")
        pltpu.sync_copy(idx_hbm.at[wid], idx_v)
        base = wid * rpw
        cp_a = pltpu.async_copy(table_hbm.at[idx_v.at[0]], rows_a, sem_a)
        cp_b = pltpu.async_copy(table_hbm.at[idx_v.at[1]], rows_b, sem_b)
        cp_a.wait()
        pltpu.sync_copy(rows_a, out_hbm.at[pl.ds(base, cpw)])
        cp_b.wait()
        pltpu.sync_copy(rows_b, out_hbm.at[pl.ds(base + cpw, cpw)])

    return sc_gather, nw, nch, cpw


def kernel(z, embedding):
    b, c, h, w = z.shape
    d = h * w
    z_flat = z.reshape(b, c, d)
    idx3, flat3, loss = _vq_tc(z_flat, embedding)
    indices_out = idx3.reshape(c, b).transpose(1, 0)
    commit_loss = jnp.sum(loss) * (_BETA / (_B * _C * _D))

    sc_gather, nw, nch, cpw = _make_sc_gather()
    table = embedding.reshape(c * _K, d)
    flat_idx = flat3.reshape(c, b).transpose(1, 0).reshape(nw, nch, cpw)
    q_rows = sc_gather(table, flat_idx)
    q_st = q_rows.reshape(b, c, h, w)
    return (q_st, commit_loss, indices_out)
